# baseline (speedup 1.0000x reference)
import functools

import jax
import jax.numpy as jnp
from jax.experimental import pallas as pl
from jax.experimental.pallas import tpu as pltpu


def _fused_kernel(x_ref, w_ref, gamma_ref, beta_ref, mean_ref, var_ref,
                  o_ref, acc_ref, *, inv_hw, eps):
    thw = x_ref.shape[1]
    hw = o_ref.shape[1]
    cout = o_ref.shape[2]
    s = pl.program_id(1)
    n_sub = pl.num_programs(1)

    xb = x_ref[0]
    ps = xb[0:8]
    for i in range(1, thw // 8):
        ps = ps + xb[i * 8:(i + 1) * 8]

    @pl.when(s == 0)
    def _init():
        acc_ref[...] = ps

    @pl.when(s > 0)
    def _accum():
        acc_ref[...] += ps

    @pl.when(s == n_sub - 1)
    def _finalize():
        row = jnp.sum(acc_ref[...], axis=0, keepdims=True)
        pooled = jax.lax.dot_general(
            row, w_ref[...], (((1,), (1,)), ((), ())),
            preferred_element_type=jnp.float32)
        scale = gamma_ref[...] * jax.lax.rsqrt(var_ref[...] + eps)
        y = pooled * (scale * inv_hw) + (beta_ref[...]
                                         - mean_ref[...] * scale)
        y = jnp.maximum(y, 0.0)
        tile = jnp.broadcast_to(y, (128, cout))
        for i in range(hw // 128):
            o_ref[0, i * 128:(i + 1) * 128, :] = tile


def kernel(x, conv_w, bn_gamma, bn_beta, bn_mean, bn_var, eps=1e-5):
    n, cin, h, w = x.shape
    cout = conv_w.shape[0]
    hw = h * w

    x_t = x.transpose(0, 2, 3, 1).reshape(n, hw, cin)

    n_sub = 2 if hw % 256 == 0 else 1
    thw = hw // n_sub

    body = functools.partial(_fused_kernel, inv_hw=1.0 / hw, eps=eps)
    out_t = pl.pallas_call(
        body,
        out_shape=jax.ShapeDtypeStruct((n, hw, cout), x.dtype),
        grid=(n, n_sub),
        in_specs=[
            pl.BlockSpec((1, thw, cin), lambda b, s: (b, s, 0)),
            pl.BlockSpec((cout, cin), lambda b, s: (0, 0)),
            pl.BlockSpec((1, cout), lambda b, s: (0, 0)),
            pl.BlockSpec((1, cout), lambda b, s: (0, 0)),
            pl.BlockSpec((1, cout), lambda b, s: (0, 0)),
            pl.BlockSpec((1, cout), lambda b, s: (0, 0)),
        ],
        out_specs=pl.BlockSpec((1, hw, cout), lambda b, s: (b, 0, 0)),
        scratch_shapes=[pltpu.VMEM((8, cin), jnp.float32)],
        compiler_params=pltpu.CompilerParams(
            dimension_semantics=("parallel", "arbitrary"),
            vmem_limit_bytes=64 * 1024 * 1024,
        ),
    )(x_t, conv_w,
      bn_gamma.reshape(1, cout), bn_beta.reshape(1, cout),
      bn_mean.reshape(1, cout), bn_var.reshape(1, cout))

    return out_t.reshape(n, h, w, cout).transpose(0, 3, 1, 2)


# speedup vs baseline: 1.1386x; 1.1386x over previous
import functools

import jax
import jax.numpy as jnp
from jax.experimental import pallas as pl
from jax.experimental.pallas import tpu as pltpu


def _fused_kernel(x_ref, w_ref, gamma_ref, beta_ref, mean_ref, var_ref,
                  o_ref, acc_ref, *, inv_hw, eps):
    thw = x_ref.shape[1]
    hw = o_ref.shape[1]
    cout = o_ref.shape[2]
    s = pl.program_id(1)
    n_sub = pl.num_programs(1)

    xb = x_ref[0]
    ps = xb[0:8]
    for i in range(1, thw // 8):
        ps = ps + xb[i * 8:(i + 1) * 8]

    @pl.when(s == 0)
    def _init():
        acc_ref[...] = ps

    @pl.when(s > 0)
    def _accum():
        acc_ref[...] += ps

    @pl.when(s == n_sub - 1)
    def _finalize():
        row = jnp.sum(acc_ref[...], axis=0, keepdims=True)
        pooled = jax.lax.dot_general(
            row, w_ref[...], (((1,), (1,)), ((), ())),
            preferred_element_type=jnp.float32)
        scale = gamma_ref[...] * jax.lax.rsqrt(var_ref[...] + eps)
        y = pooled * (scale * inv_hw) + (beta_ref[...]
                                         - mean_ref[...] * scale)
        y = jnp.maximum(y, 0.0)
        tile = jnp.broadcast_to(y, (128, cout))
        for i in range(hw // 128):
            o_ref[0, i * 128:(i + 1) * 128, :] = tile


def kernel(x, conv_w, bn_gamma, bn_beta, bn_mean, bn_var, eps=1e-5):
    n, cin, h, w = x.shape
    cout = conv_w.shape[0]
    hw = h * w

    x_t = x.transpose(0, 2, 3, 1).reshape(n, hw, cin)

    n_sub = 1
    thw = hw // n_sub

    body = functools.partial(_fused_kernel, inv_hw=1.0 / hw, eps=eps)
    out_t = pl.pallas_call(
        body,
        out_shape=jax.ShapeDtypeStruct((n, hw, cout), x.dtype),
        grid=(n, n_sub),
        in_specs=[
            pl.BlockSpec((1, thw, cin), lambda b, s: (b, s, 0)),
            pl.BlockSpec((cout, cin), lambda b, s: (0, 0)),
            pl.BlockSpec((1, cout), lambda b, s: (0, 0)),
            pl.BlockSpec((1, cout), lambda b, s: (0, 0)),
            pl.BlockSpec((1, cout), lambda b, s: (0, 0)),
            pl.BlockSpec((1, cout), lambda b, s: (0, 0)),
        ],
        out_specs=pl.BlockSpec((1, hw, cout), lambda b, s: (b, 0, 0)),
        scratch_shapes=[pltpu.VMEM((8, cin), jnp.float32)],
        compiler_params=pltpu.CompilerParams(
            dimension_semantics=("parallel", "arbitrary"),
            vmem_limit_bytes=64 * 1024 * 1024,
        ),
    )(x_t, conv_w,
      bn_gamma.reshape(1, cout), bn_beta.reshape(1, cout),
      bn_mean.reshape(1, cout), bn_var.reshape(1, cout))

    return out_t.reshape(n, h, w, cout).transpose(0, 3, 1, 2)


# speedup vs baseline: 1.1409x; 1.0020x over previous
import functools

import jax
import jax.numpy as jnp
from jax.experimental import pallas as pl
from jax.experimental.pallas import tpu as pltpu


def _fused_kernel(x_ref, w_ref, gamma_ref, beta_ref, mean_ref, var_ref,
                  o_ref, *, inv_hw, eps):
    hw = x_ref.shape[1]
    cout = o_ref.shape[2]

    xb = x_ref[0]
    ps = xb[0:8]
    for i in range(1, hw // 8):
        ps = ps + xb[i * 8:(i + 1) * 8]
    row = jnp.sum(ps, axis=0, keepdims=True)

    pooled = jax.lax.dot_general(
        row, w_ref[...], (((1,), (1,)), ((), ())),
        preferred_element_type=jnp.float32)

    scale = gamma_ref[...] * jax.lax.rsqrt(var_ref[...] + eps)
    y = pooled * (scale * inv_hw) + (beta_ref[...] - mean_ref[...] * scale)
    y = jnp.maximum(y, 0.0)

    tile = jnp.broadcast_to(y, (128, cout))
    for i in range(hw // 128):
        o_ref[0, i * 128:(i + 1) * 128, :] = tile


def kernel(x, conv_w, bn_gamma, bn_beta, bn_mean, bn_var, eps=1e-5):
    n, cin, h, w = x.shape
    cout = conv_w.shape[0]
    hw = h * w

    x_t = x.transpose(0, 2, 3, 1).reshape(n, hw, cin)

    body = functools.partial(_fused_kernel, inv_hw=1.0 / hw, eps=eps)
    out_t = pl.pallas_call(
        body,
        out_shape=jax.ShapeDtypeStruct((n, hw, cout), x.dtype),
        grid=(n,),
        in_specs=[
            pl.BlockSpec((1, hw, cin), lambda b: (b, 0, 0)),
            pl.BlockSpec((cout, cin), lambda b: (0, 0)),
            pl.BlockSpec((1, cout), lambda b: (0, 0)),
            pl.BlockSpec((1, cout), lambda b: (0, 0)),
            pl.BlockSpec((1, cout), lambda b: (0, 0)),
            pl.BlockSpec((1, cout), lambda b: (0, 0)),
        ],
        out_specs=pl.BlockSpec((1, hw, cout), lambda b: (b, 0, 0)),
        compiler_params=pltpu.CompilerParams(
            dimension_semantics=("parallel",),
            vmem_limit_bytes=64 * 1024 * 1024,
        ),
    )(x_t, conv_w,
      bn_gamma.reshape(1, cout), bn_beta.reshape(1, cout),
      bn_mean.reshape(1, cout), bn_var.reshape(1, cout))

    return out_t.reshape(n, h, w, cout).transpose(0, 3, 1, 2)
